# baseline (speedup 1.0000x reference)
import functools

import jax
import jax.numpy as jnp
from jax import lax
from jax.experimental import pallas as pl
from jax.experimental.pallas import tpu as pltpu
from jax.experimental.pallas import tpu_sc as plsc

N_NODES = 10000
N_PAD = 10240
D_FEAT = 128
N_EDGES = 320000

NC = 2
NS = 16
NW = NC * NS
CHUNK = 128
K = 2
GROUPS = 40
N_CHUNKS = GROUPS * K
EDGES_PER_TILE = N_CHUNKS * CHUNK
E_TOTAL = NW * EDGES_PER_TILE
N_EDGE_PAD = E_TOTAL - N_EDGES
ROWS_PER_TILE = N_PAD // NS
ZBUF_ROWS = 40

ROW_BLK = 1280


def _sc_segment_sum(features, src, dst):
    mesh = plsc.VectorSubcoreMesh(core_axis_name="c", subcore_axis_name="s")

    @functools.partial(
        pl.kernel,
        out_type=jax.ShapeDtypeStruct((NC, N_PAD, D_FEAT), jnp.float32),
        mesh=mesh,
        scratch_types=(
            [pltpu.VMEM((CHUNK,), jnp.int32)] * K
            + [pltpu.VMEM((CHUNK,), jnp.int32)] * K
            + [pltpu.VMEM((CHUNK, D_FEAT), jnp.float32)] * K
            + [
                pltpu.VMEM((ZBUF_ROWS, D_FEAT), jnp.float32),
                pltpu.VMEM_SHARED((N_PAD, D_FEAT), jnp.float32),
            ]
            + [pltpu.SemaphoreType.DMA] * K
            + [pltpu.SemaphoreType.DMA] * K
            + [pltpu.SemaphoreType.DMA] * K
        ),
    )
    def sc_kernel(feat_hbm, src_hbm, dst_hbm, out_hbm, *scr):
        idx_s = scr[:K]
        idx_d = scr[K:2 * K]
        rows = scr[2 * K:3 * K]
        zbuf = scr[3 * K]
        acc = scr[3 * K + 1]
        isems = scr[3 * K + 2:3 * K + 2 + K]
        gsems = scr[3 * K + 2 + K:3 * K + 2 + 2 * K]
        ssems = scr[3 * K + 2 + 2 * K:]
        cid = lax.axis_index("c")
        sid = lax.axis_index("s")
        wid = cid * NS + sid

        ebase = wid * EDGES_PER_TILE

        def fire_idx(k, j):
            off = ebase + j * CHUNK
            pltpu.async_copy(src_hbm.at[pl.ds(off, CHUNK)], idx_s[k], isems[k])
            pltpu.async_copy(dst_hbm.at[pl.ds(off, CHUNK)], idx_d[k], isems[k])

        def wait_idx(k, j):
            off = ebase + j * CHUNK
            pltpu.make_async_copy(src_hbm.at[pl.ds(off, CHUNK)], idx_s[k],
                                  isems[k]).wait()
            pltpu.make_async_copy(dst_hbm.at[pl.ds(off, CHUNK)], idx_d[k],
                                  isems[k]).wait()

        def fire_gather(k):
            pltpu.async_copy(feat_hbm.at[idx_s[k]], rows[k], gsems[k])

        def wait_gather(k):
            pltpu.make_async_copy(feat_hbm.at[idx_s[k]], rows[k],
                                  gsems[k]).wait()

        def fire_scatter(k):
            pltpu.async_copy(rows[k], acc.at[idx_d[k]], ssems[k], add=True)

        def wait_scatter(k):
            pltpu.make_async_copy(rows[k], acc.at[idx_d[k]],
                                  ssems[k]).wait()

        for r in range(ZBUF_ROWS):
            for c in range(D_FEAT // 16):
                zbuf[r, pl.ds(c * 16, 16)] = jnp.zeros((16,), jnp.float32)

        row0 = sid * ROWS_PER_TILE
        zsem = ssems[0]
        for i in range(ROWS_PER_TILE // ZBUF_ROWS):
            pltpu.async_copy(
                zbuf, acc.at[pl.ds(row0 + i * ZBUF_ROWS, ZBUF_ROWS)], zsem)

        for k in range(K):
            fire_idx(k, k)
        for k in range(K):
            wait_idx(k, k)
            fire_gather(k)

        for i in range(ROWS_PER_TILE // ZBUF_ROWS):
            pltpu.make_async_copy(
                zbuf, acc.at[pl.ds(row0 + i * ZBUF_ROWS, ZBUF_ROWS)], zsem).wait()
        plsc.subcore_barrier()

        @pl.loop(0, GROUPS)
        def _group(g):
            base = g * K
            for k in range(K):
                wait_gather(k)
                fire_scatter(k)
            for k in range(K):
                wait_scatter(k)
                fire_idx(k, base + K + k)
            for k in range(K):
                wait_idx(k, base + K + k)
                fire_gather(k)

        for k in range(K):
            wait_gather(k)

        plsc.subcore_barrier()
        pltpu.sync_copy(acc.at[pl.ds(row0, ROWS_PER_TILE)],
                        out_hbm.at[cid, pl.ds(row0, ROWS_PER_TILE)])

    return sc_kernel(features, src, dst)


def _tc_linear_relu(parts, W, b):
    def body(p_ref, w_ref, b_ref, o_ref):
        h = p_ref[0] + p_ref[1]
        y = lax.dot_general(h, w_ref[...], (((1,), (0,)), ((), ())),
                            preferred_element_type=jnp.float32,
                            precision=lax.Precision.HIGHEST)
        o_ref[...] = jnp.maximum(y + b_ref[...], 0.0)

    return pl.pallas_call(
        body,
        grid=(N_PAD // ROW_BLK,),
        in_specs=[
            pl.BlockSpec((NC, ROW_BLK, D_FEAT), lambda i: (0, i, 0)),
            pl.BlockSpec((D_FEAT, D_FEAT), lambda i: (0, 0)),
            pl.BlockSpec((1, D_FEAT), lambda i: (0, 0)),
        ],
        out_specs=pl.BlockSpec((ROW_BLK, D_FEAT), lambda i: (i, 0)),
        out_shape=jax.ShapeDtypeStruct((N_PAD, D_FEAT), jnp.float32),
    )(parts, W, b.reshape(1, D_FEAT))


def kernel(features, edge_index, W, b):
    extra = K * CHUNK
    pad_iota = jnp.arange(N_EDGE_PAD + extra, dtype=jnp.int32)
    pad_src = pad_iota % N_NODES
    pad_dst = N_NODES + pad_iota % (N_PAD - N_NODES)
    src = jnp.concatenate([edge_index[0].astype(jnp.int32), pad_src])
    dst = jnp.concatenate([edge_index[1].astype(jnp.int32), pad_dst])
    parts = _sc_segment_sum(features, src, dst)
    return _tc_linear_relu(parts, W, b)[:N_NODES]


# speedup vs baseline: 1.1375x; 1.1375x over previous
import functools

import jax
import jax.numpy as jnp
from jax import lax
from jax.experimental import pallas as pl
from jax.experimental.pallas import tpu as pltpu
from jax.experimental.pallas import tpu_sc as plsc

N_NODES = 10000
N_PAD = 10240
D_FEAT = 128
N_EDGES = 320000

NC = 2
NS = 16
NW = NC * NS
CHUNK = 64
K = 4
GROUPS = 40
N_CHUNKS = GROUPS * K
EDGES_PER_TILE = N_CHUNKS * CHUNK
E_TOTAL = NW * EDGES_PER_TILE
N_EDGE_PAD = E_TOTAL - N_EDGES
ROWS_PER_TILE = N_PAD // NS
ZBUF_ROWS = 40

ROW_BLK = 1280


def _sc_segment_sum(features, src, dst):
    mesh = plsc.VectorSubcoreMesh(core_axis_name="c", subcore_axis_name="s")

    @functools.partial(
        pl.kernel,
        out_type=jax.ShapeDtypeStruct((NC, N_PAD, D_FEAT), jnp.float32),
        mesh=mesh,
        scratch_types=(
            [pltpu.VMEM((CHUNK,), jnp.int32)] * K
            + [pltpu.VMEM((CHUNK,), jnp.int32)] * K
            + [pltpu.VMEM((CHUNK, D_FEAT), jnp.float32)] * K
            + [
                pltpu.VMEM((ZBUF_ROWS, D_FEAT), jnp.float32),
                pltpu.VMEM_SHARED((N_PAD, D_FEAT), jnp.float32),
            ]
            + [pltpu.SemaphoreType.DMA] * K
            + [pltpu.SemaphoreType.DMA] * K
            + [pltpu.SemaphoreType.DMA] * K
        ),
    )
    def sc_kernel(feat_hbm, src_hbm, dst_hbm, out_hbm, *scr):
        idx_s = scr[:K]
        idx_d = scr[K:2 * K]
        rows = scr[2 * K:3 * K]
        zbuf = scr[3 * K]
        acc = scr[3 * K + 1]
        isems = scr[3 * K + 2:3 * K + 2 + K]
        gsems = scr[3 * K + 2 + K:3 * K + 2 + 2 * K]
        ssems = scr[3 * K + 2 + 2 * K:]
        cid = lax.axis_index("c")
        sid = lax.axis_index("s")
        wid = cid * NS + sid

        ebase = wid * EDGES_PER_TILE

        def fire_idx(k, j):
            off = ebase + j * CHUNK
            pltpu.async_copy(src_hbm.at[pl.ds(off, CHUNK)], idx_s[k], isems[k])
            pltpu.async_copy(dst_hbm.at[pl.ds(off, CHUNK)], idx_d[k], isems[k])

        def wait_idx(k, j):
            off = ebase + j * CHUNK
            pltpu.make_async_copy(src_hbm.at[pl.ds(off, CHUNK)], idx_s[k],
                                  isems[k]).wait()
            pltpu.make_async_copy(dst_hbm.at[pl.ds(off, CHUNK)], idx_d[k],
                                  isems[k]).wait()

        def fire_gather(k):
            pltpu.async_copy(feat_hbm.at[idx_s[k]], rows[k], gsems[k])

        def wait_gather(k):
            pltpu.make_async_copy(feat_hbm.at[idx_s[k]], rows[k],
                                  gsems[k]).wait()

        def fire_scatter(k):
            pltpu.async_copy(rows[k], acc.at[idx_d[k]], ssems[k], add=True)

        def wait_scatter(k):
            pltpu.make_async_copy(rows[k], acc.at[idx_d[k]],
                                  ssems[k]).wait()

        for r in range(ZBUF_ROWS):
            for c in range(D_FEAT // 16):
                zbuf[r, pl.ds(c * 16, 16)] = jnp.zeros((16,), jnp.float32)

        row0 = sid * ROWS_PER_TILE
        zsem = ssems[0]
        for i in range(ROWS_PER_TILE // ZBUF_ROWS):
            pltpu.async_copy(
                zbuf, acc.at[pl.ds(row0 + i * ZBUF_ROWS, ZBUF_ROWS)], zsem)

        for k in range(K):
            fire_idx(k, k)
        for k in range(K):
            wait_idx(k, k)
            fire_gather(k)

        for i in range(ROWS_PER_TILE // ZBUF_ROWS):
            pltpu.make_async_copy(
                zbuf, acc.at[pl.ds(row0 + i * ZBUF_ROWS, ZBUF_ROWS)], zsem).wait()
        plsc.subcore_barrier()

        @pl.loop(0, GROUPS)
        def _group(g):
            base = g * K
            for k in range(K):
                wait_gather(k)
                fire_scatter(k)
            for k in range(K):
                wait_scatter(k)
                fire_idx(k, base + K + k)
            for k in range(K):
                wait_idx(k, base + K + k)
                fire_gather(k)

        for k in range(K):
            wait_gather(k)

        plsc.subcore_barrier()
        pltpu.sync_copy(acc.at[pl.ds(row0, ROWS_PER_TILE)],
                        out_hbm.at[cid, pl.ds(row0, ROWS_PER_TILE)])

    return sc_kernel(features, src, dst)


def _tc_linear_relu(parts, W, b):
    def body(p_ref, w_ref, b_ref, o_ref):
        h = p_ref[0] + p_ref[1]
        y = lax.dot_general(h, w_ref[...], (((1,), (0,)), ((), ())),
                            preferred_element_type=jnp.float32,
                            precision=lax.Precision.HIGHEST)
        o_ref[...] = jnp.maximum(y + b_ref[...], 0.0)

    return pl.pallas_call(
        body,
        grid=(N_PAD // ROW_BLK,),
        in_specs=[
            pl.BlockSpec((NC, ROW_BLK, D_FEAT), lambda i: (0, i, 0)),
            pl.BlockSpec((D_FEAT, D_FEAT), lambda i: (0, 0)),
            pl.BlockSpec((1, D_FEAT), lambda i: (0, 0)),
        ],
        out_specs=pl.BlockSpec((ROW_BLK, D_FEAT), lambda i: (i, 0)),
        out_shape=jax.ShapeDtypeStruct((N_PAD, D_FEAT), jnp.float32),
    )(parts, W, b.reshape(1, D_FEAT))


def kernel(features, edge_index, W, b):
    extra = K * CHUNK
    pad_iota = jnp.arange(N_EDGE_PAD + extra, dtype=jnp.int32)
    pad_src = pad_iota % N_NODES
    pad_dst = N_NODES + pad_iota % (N_PAD - N_NODES)
    src = jnp.concatenate([edge_index[0].astype(jnp.int32), pad_src])
    dst = jnp.concatenate([edge_index[1].astype(jnp.int32), pad_dst])
    parts = _sc_segment_sum(features, src, dst)
    return _tc_linear_relu(parts, W, b)[:N_NODES]


# speedup vs baseline: 1.1493x; 1.0104x over previous
import functools

import jax
import jax.numpy as jnp
from jax import lax
from jax.experimental import pallas as pl
from jax.experimental.pallas import tpu as pltpu
from jax.experimental.pallas import tpu_sc as plsc

N_NODES = 10000
N_PAD = 10240
D_FEAT = 128
N_EDGES = 320000

NC = 2
NS = 16
NW = NC * NS
CHUNK = 40
K = 6
GROUPS = 42
N_CHUNKS = GROUPS * K
EDGES_PER_TILE = N_CHUNKS * CHUNK
E_TOTAL = NW * EDGES_PER_TILE
N_EDGE_PAD = E_TOTAL - N_EDGES
ROWS_PER_TILE = N_PAD // NS
ZBUF_ROWS = 40

ROW_BLK = 1280


def _sc_segment_sum(features, src, dst):
    mesh = plsc.VectorSubcoreMesh(core_axis_name="c", subcore_axis_name="s")

    @functools.partial(
        pl.kernel,
        out_type=jax.ShapeDtypeStruct((NC, N_PAD, D_FEAT), jnp.float32),
        mesh=mesh,
        scratch_types=(
            [pltpu.VMEM((CHUNK,), jnp.int32)] * K
            + [pltpu.VMEM((CHUNK,), jnp.int32)] * K
            + [pltpu.VMEM((CHUNK, D_FEAT), jnp.float32)] * K
            + [
                pltpu.VMEM((ZBUF_ROWS, D_FEAT), jnp.float32),
                pltpu.VMEM_SHARED((N_PAD, D_FEAT), jnp.float32),
            ]
            + [pltpu.SemaphoreType.DMA] * K
            + [pltpu.SemaphoreType.DMA] * K
            + [pltpu.SemaphoreType.DMA] * K
        ),
    )
    def sc_kernel(feat_hbm, src_hbm, dst_hbm, out_hbm, *scr):
        idx_s = scr[:K]
        idx_d = scr[K:2 * K]
        rows = scr[2 * K:3 * K]
        zbuf = scr[3 * K]
        acc = scr[3 * K + 1]
        isems = scr[3 * K + 2:3 * K + 2 + K]
        gsems = scr[3 * K + 2 + K:3 * K + 2 + 2 * K]
        ssems = scr[3 * K + 2 + 2 * K:]
        cid = lax.axis_index("c")
        sid = lax.axis_index("s")
        wid = cid * NS + sid

        ebase = wid * EDGES_PER_TILE

        def fire_idx(k, j):
            off = ebase + j * CHUNK
            pltpu.async_copy(src_hbm.at[pl.ds(off, CHUNK)], idx_s[k], isems[k])
            pltpu.async_copy(dst_hbm.at[pl.ds(off, CHUNK)], idx_d[k], isems[k])

        def wait_idx(k, j):
            off = ebase + j * CHUNK
            pltpu.make_async_copy(src_hbm.at[pl.ds(off, CHUNK)], idx_s[k],
                                  isems[k]).wait()
            pltpu.make_async_copy(dst_hbm.at[pl.ds(off, CHUNK)], idx_d[k],
                                  isems[k]).wait()

        def fire_gather(k):
            pltpu.async_copy(feat_hbm.at[idx_s[k]], rows[k], gsems[k])

        def wait_gather(k):
            pltpu.make_async_copy(feat_hbm.at[idx_s[k]], rows[k],
                                  gsems[k]).wait()

        def fire_scatter(k):
            pltpu.async_copy(rows[k], acc.at[idx_d[k]], ssems[k], add=True)

        def wait_scatter(k):
            pltpu.make_async_copy(rows[k], acc.at[idx_d[k]],
                                  ssems[k]).wait()

        for r in range(ZBUF_ROWS):
            for c in range(D_FEAT // 16):
                zbuf[r, pl.ds(c * 16, 16)] = jnp.zeros((16,), jnp.float32)

        row0 = sid * ROWS_PER_TILE
        zsem = ssems[0]
        for i in range(ROWS_PER_TILE // ZBUF_ROWS):
            pltpu.async_copy(
                zbuf, acc.at[pl.ds(row0 + i * ZBUF_ROWS, ZBUF_ROWS)], zsem)

        for k in range(K):
            fire_idx(k, k)
        for k in range(K):
            wait_idx(k, k)
            fire_gather(k)

        for i in range(ROWS_PER_TILE // ZBUF_ROWS):
            pltpu.make_async_copy(
                zbuf, acc.at[pl.ds(row0 + i * ZBUF_ROWS, ZBUF_ROWS)], zsem).wait()
        plsc.subcore_barrier()

        @pl.loop(0, GROUPS)
        def _group(g):
            base = g * K
            for k in range(K):
                wait_gather(k)
                fire_scatter(k)
            for k in range(K):
                wait_scatter(k)
                fire_idx(k, base + K + k)
            for k in range(K):
                wait_idx(k, base + K + k)
                fire_gather(k)

        for k in range(K):
            wait_gather(k)

        plsc.subcore_barrier()
        pltpu.sync_copy(acc.at[pl.ds(row0, ROWS_PER_TILE)],
                        out_hbm.at[cid, pl.ds(row0, ROWS_PER_TILE)])

    return sc_kernel(features, src, dst)


def _tc_linear_relu(parts, W, b):
    def body(p_ref, w_ref, b_ref, o_ref):
        h = p_ref[0] + p_ref[1]
        y = lax.dot_general(h, w_ref[...], (((1,), (0,)), ((), ())),
                            preferred_element_type=jnp.float32,
                            precision=lax.Precision.HIGHEST)
        o_ref[...] = jnp.maximum(y + b_ref[...], 0.0)

    return pl.pallas_call(
        body,
        grid=(N_PAD // ROW_BLK,),
        in_specs=[
            pl.BlockSpec((NC, ROW_BLK, D_FEAT), lambda i: (0, i, 0)),
            pl.BlockSpec((D_FEAT, D_FEAT), lambda i: (0, 0)),
            pl.BlockSpec((1, D_FEAT), lambda i: (0, 0)),
        ],
        out_specs=pl.BlockSpec((ROW_BLK, D_FEAT), lambda i: (i, 0)),
        out_shape=jax.ShapeDtypeStruct((N_PAD, D_FEAT), jnp.float32),
    )(parts, W, b.reshape(1, D_FEAT))


def kernel(features, edge_index, W, b):
    extra = K * CHUNK
    pad_iota = jnp.arange(N_EDGE_PAD + extra, dtype=jnp.int32)
    pad_src = pad_iota % N_NODES
    pad_dst = N_NODES + pad_iota % (N_PAD - N_NODES)
    src = jnp.concatenate([edge_index[0].astype(jnp.int32), pad_src])
    dst = jnp.concatenate([edge_index[1].astype(jnp.int32), pad_dst])
    parts = _sc_segment_sum(features, src, dst)
    return _tc_linear_relu(parts, W, b)[:N_NODES]
